# Initial kernel scaffold; baseline (speedup 1.0000x reference)
import jax
import jax.numpy as jnp
from jax.experimental import pallas as pl


def kernel(input_, offsets, emb_weight, proj_w, proj_b, att_h):
    raise NotImplementedError("write your pallas kernel here")


# speedup vs baseline: 23.5438x; 23.5438x over previous
import functools

import jax
import jax.numpy as jnp
from jax import lax
from jax.experimental import pallas as pl
from jax.experimental.pallas import tpu as pltpu
from jax.experimental.pallas import tpu_sc as plsc

_NC = 2
_NS = 16
_NW = _NC * _NS
_L = 16

_BLK = 128
_NEG = -3e38


def _att_table_tc(emb_weight, proj_w, proj_b, att_h):
  V, D = emb_weight.shape
  A = proj_w.shape[0]
  RB = 1024
  grid = (V + RB - 1) // RB

  pwt = proj_w.T
  pb = jnp.broadcast_to(proj_b[None, :], (8, A))
  ah = jnp.broadcast_to(att_h[None, :], (8, A))

  def body(emb_ref, pwt_ref, pb_ref, ah_ref, out_ref):
    x = emb_ref[...]
    t = jnp.tanh(
        jnp.dot(x, pwt_ref[...], preferred_element_type=jnp.float32)
        + pb_ref[0:1, :])
    s = jnp.sum(t * ah_ref[0:1, :], axis=1)
    out_ref[...] = s.reshape(1, 1, RB)

  out = pl.pallas_call(
      body,
      grid=(grid,),
      in_specs=[
          pl.BlockSpec((RB, D), lambda i: (i, 0)),
          pl.BlockSpec((D, A), lambda i: (0, 0)),
          pl.BlockSpec((8, A), lambda i: (0, 0)),
          pl.BlockSpec((8, A), lambda i: (0, 0)),
      ],
      out_specs=pl.BlockSpec((1, 1, RB), lambda i: (i, 0, 0)),
      out_shape=jax.ShapeDtypeStruct((grid, 1, RB), jnp.float32),
  )(emb_weight, pwt, pb, ah)
  return out.reshape(grid * RB)


def _pooled_sc(input_, offs_pad, vocab_att, emb_weight, B):
  NNZ = input_.shape[0]
  V, D = emb_weight.shape
  bags_w = B // _NW
  ND = D // _L
  VT = ((V + _L - 1) // _L) * _L
  mesh = plsc.VectorSubcoreMesh(core_axis_name="c", subcore_axis_name="s")

  @functools.partial(
      pl.kernel,
      mesh=mesh,
      out_type=jax.ShapeDtypeStruct((B, D), jnp.float32),
      scratch_types=[
          pltpu.VMEM((VT,), jnp.float32),
          pltpu.VMEM((bags_w + 2 * _L,), jnp.int32),
          pltpu.VMEM((_BLK,), jnp.int32),
          pltpu.VMEM((_BLK,), jnp.float32),
          pltpu.VMEM((_BLK, D), jnp.float32),
          pltpu.VMEM((bags_w, D), jnp.float32),
      ],
      compiler_params=pltpu.CompilerParams(
          needs_layout_passes=False, use_tc_tiling_on_sc=False),
  )
  def k(input_hbm, offs_hbm, vatt_hbm, emb_hbm, out_hbm,
        table, offs_v, cols_v, att_v, rows_v, outb_v):
    wid = lax.axis_index("s") * _NC + lax.axis_index("c")
    bag0 = wid * bags_w
    lanes = lax.iota(jnp.int32, _L)
    ones = jnp.full((_L,), 1.0, jnp.float32)
    zeros = jnp.zeros((_L,), jnp.float32)

    def _zt(i, c):
      table[pl.ds(i * _L, _L)] = zeros
      return c
    lax.fori_loop(0, VT // _L, _zt, 0)

    pltpu.sync_copy(
        offs_hbm.at[pl.ds(pl.multiple_of(bag0, 8), bags_w + 2 * _L)], offs_v)

    def bag_body(bi, c):
      win0 = pl.multiple_of((bi >> 3) << 3, 8)
      win = offs_v[pl.ds(win0, _L)]
      loc = bi - win0
      start = jnp.max(jnp.where(lanes == loc, win, jnp.int32(-2147483647)))
      end = jnp.max(jnp.where(lanes == loc + 1, win, jnp.int32(-2147483647)))
      base0 = (start >> 7) << 7
      nblk = (end - base0 + _BLK - 1) >> 7

      def stage_cols(blk):
        base = pl.multiple_of(base0 + blk * _BLK, _BLK)
        pltpu.sync_copy(input_hbm.at[pl.ds(base, _BLK)], cols_v)

      def chunk(blk, si):
        base = base0 + blk * _BLK + si * _L
        cv = cols_v[pl.ds(si * _L, _L)]
        gidx = base + lanes
        valid = (gidx >= start) & (gidx < end)
        return cv, valid

      def p1(blk, c):
        stage_cols(blk)
        for si in range(_BLK // _L):
          cv, valid = chunk(blk, si)
          for j in range(_L):
            plsc.addupdate_scatter(table, [cv], ones,
                                   mask=valid & (lanes == j))
        return c
      lax.fori_loop(0, nblk, p1, 0)

      def p2(blk, carry):
        m_vec, s_vec = carry
        stage_cols(blk)
        pltpu.sync_copy(vatt_hbm.at[cols_v], att_v)
        for si in range(_BLK // _L):
          cv, valid = chunk(blk, si)
          cnt = plsc.load_gather(table, [cv])
          cnt = jnp.where(valid, cnt, 1.0)
          a = att_v[pl.ds(si * _L, _L)]
          g = jnp.where(valid, cnt * a, _NEG)
          m_new = jnp.maximum(m_vec, g)
          e = jnp.where(valid, jnp.exp(g - m_new) / cnt, 0.0)
          s_vec = s_vec * jnp.exp(m_vec - m_new) + e
          m_vec = m_new
        return m_vec, s_vec

      m_vec, s_vec = lax.fori_loop(
          0, nblk, p2, (jnp.full((_L,), _NEG), jnp.zeros((_L,), jnp.float32)))
      m_b = jnp.max(m_vec)
      s_b = jnp.sum(s_vec * jnp.exp(m_vec - m_b))
      s_bv = jnp.full((_L,), s_b, jnp.float32)
      inv_s = ones / jnp.where(s_bv > 0.0, s_bv, 1.0)

      def p3(blk, acc):
        stage_cols(blk)
        pltpu.sync_copy(vatt_hbm.at[cols_v], att_v)
        pltpu.sync_copy(emb_hbm.at[cols_v], rows_v)
        for si in range(_BLK // _L):
          cv, valid = chunk(blk, si)
          cnt = plsc.load_gather(table, [cv])
          cnt = jnp.where(valid, cnt, 1.0)
          a = att_v[pl.ds(si * _L, _L)]
          g = cnt * a
          w = jnp.where(valid, jnp.exp(g - m_b) * (inv_s / cnt), 0.0)
          for j in range(_L):
            wj = w[j]
            tok = si * _L + j
            acc = tuple(
                acc[d] + wj * rows_v[tok, pl.ds(d * _L, _L)]
                for d in range(ND))
        return acc
      acc = lax.fori_loop(
          0, nblk, p3, tuple(jnp.zeros((_L,), jnp.float32) for _ in range(ND)))
      for d in range(ND):
        outb_v[bi, pl.ds(d * _L, _L)] = acc[d]

      def p4(blk, c):
        stage_cols(blk)
        for si in range(_BLK // _L):
          cv, valid = chunk(blk, si)
          plsc.store_scatter(table, [cv], zeros, mask=valid)
        return c
      lax.fori_loop(0, nblk, p4, 0)
      return c

    lax.fori_loop(0, bags_w, bag_body, 0)
    pltpu.sync_copy(outb_v, out_hbm.at[pl.ds(pl.multiple_of(bag0, 8), bags_w)])

  return k(input_, offs_pad, vocab_att, emb_weight)


def kernel(input_, offsets, emb_weight, proj_w, proj_b, att_h):
  B = offsets.shape[0]
  NNZ = input_.shape[0]
  vocab_att = _att_table_tc(emb_weight, proj_w, proj_b, att_h)
  offs_pad = jnp.concatenate(
      [offsets, jnp.full((_NW * 8,), NNZ, dtype=offsets.dtype)])
  return _pooled_sc(input_, offs_pad, vocab_att, emb_weight, B)


# speedup vs baseline: 27.2577x; 1.1577x over previous
import functools

import jax
import jax.numpy as jnp
from jax import lax
from jax.experimental import pallas as pl
from jax.experimental.pallas import tpu as pltpu
from jax.experimental.pallas import tpu_sc as plsc

_NC = 2
_NS = 16
_NW = _NC * _NS
_L = 16

_BLK = 128
_NEG = -3e38


def _att_table_tc(emb_weight, proj_w, proj_b, att_h):
  V, D = emb_weight.shape
  A = proj_w.shape[0]
  RB = 1024
  grid = (V + RB - 1) // RB

  pwt = proj_w.T
  pb = jnp.broadcast_to(proj_b[None, :], (8, A))
  ah = jnp.broadcast_to(att_h[None, :], (8, A))

  def body(emb_ref, pwt_ref, pb_ref, ah_ref, out_ref):
    x = emb_ref[...]
    t = jnp.tanh(
        jnp.dot(x, pwt_ref[...], preferred_element_type=jnp.float32)
        + pb_ref[0:1, :])
    s = jnp.sum(t * ah_ref[0:1, :], axis=1)
    out_ref[...] = s.reshape(1, 1, RB)

  out = pl.pallas_call(
      body,
      grid=(grid,),
      in_specs=[
          pl.BlockSpec((RB, D), lambda i: (i, 0)),
          pl.BlockSpec((D, A), lambda i: (0, 0)),
          pl.BlockSpec((8, A), lambda i: (0, 0)),
          pl.BlockSpec((8, A), lambda i: (0, 0)),
      ],
      out_specs=pl.BlockSpec((1, 1, RB), lambda i: (i, 0, 0)),
      out_shape=jax.ShapeDtypeStruct((grid, 1, RB), jnp.float32),
  )(emb_weight, pwt, pb, ah)
  return out.reshape(grid * RB)


def _pooled_sc(input_, offs_pad, vocab_att, emb_weight, B):
  NNZ = input_.shape[0]
  V, D = emb_weight.shape
  bags_w = B // _NW
  ND = D // _L
  VT = ((V + _L - 1) // _L) * _L
  mesh = plsc.VectorSubcoreMesh(core_axis_name="c", subcore_axis_name="s")

  @functools.partial(
      pl.kernel,
      mesh=mesh,
      out_type=jax.ShapeDtypeStruct((B, D), jnp.float32),
      scratch_types=[
          pltpu.VMEM((VT,), jnp.float32),
          pltpu.VMEM((bags_w + 2 * _L,), jnp.int32),
          pltpu.VMEM((_BLK,), jnp.int32),
          pltpu.VMEM((_BLK,), jnp.float32),
          pltpu.VMEM((_BLK, D), jnp.float32),
          pltpu.VMEM((bags_w, D), jnp.float32),
      ],
      compiler_params=pltpu.CompilerParams(
          needs_layout_passes=False, use_tc_tiling_on_sc=False),
  )
  def k(input_hbm, offs_hbm, vatt_hbm, emb_hbm, out_hbm,
        table, offs_v, cols_v, att_v, rows_v, outb_v):
    wid = lax.axis_index("s") * _NC + lax.axis_index("c")
    bag0 = wid * bags_w
    lanes = lax.iota(jnp.int32, _L)
    ones = jnp.full((_L,), 1.0, jnp.float32)
    zeros = jnp.zeros((_L,), jnp.float32)

    def _zt(i, c):
      table[pl.ds(i * _L, _L)] = zeros
      return c
    lax.fori_loop(0, VT // _L, _zt, 0)

    pltpu.sync_copy(
        offs_hbm.at[pl.ds(pl.multiple_of(bag0, 8), bags_w + 2 * _L)], offs_v)

    def bag_body(bi, c):
      win0 = pl.multiple_of((bi >> 3) << 3, 8)
      win = offs_v[pl.ds(win0, _L)]
      loc = bi - win0
      start = jnp.max(jnp.where(lanes == loc, win, jnp.int32(-2147483647)))
      end = jnp.max(jnp.where(lanes == loc + 1, win, jnp.int32(-2147483647)))
      base0 = (start >> 7) << 7
      nblk = (end - base0 + _BLK - 1) >> 7

      def stage_cols(blk):
        base = pl.multiple_of(base0 + blk * _BLK, _BLK)
        pltpu.sync_copy(input_hbm.at[pl.ds(base, _BLK)], cols_v)

      def chunk(blk, si):
        base = base0 + blk * _BLK + si * _L
        cv = cols_v[pl.ds(si * _L, _L)]
        gidx = base + lanes
        valid = (gidx >= start) & (gidx < end)
        return cv, valid

      def p1(blk, c):
        stage_cols(blk)
        for si in range(_BLK // _L):
          cv, valid = chunk(blk, si)
          for j in range(_L):
            plsc.addupdate_scatter(table, [cv], ones,
                                   mask=valid & (lanes == j))
        return c
      lax.fori_loop(0, nblk, p1, 0)

      multi = nblk > 1

      def p2(blk, carry):
        m_vec, s_vec = carry
        @pl.when(multi)
        def _():
          stage_cols(blk)
        pltpu.sync_copy(vatt_hbm.at[cols_v], att_v)
        for si in range(_BLK // _L):
          cv, valid = chunk(blk, si)
          cnt = plsc.load_gather(table, [cv])
          cnt = jnp.where(valid, cnt, 1.0)
          a = att_v[pl.ds(si * _L, _L)]
          g = jnp.where(valid, cnt * a, _NEG)
          m_new = jnp.maximum(m_vec, g)
          e = jnp.where(valid, jnp.exp(g - m_new) / cnt, 0.0)
          s_vec = s_vec * jnp.exp(m_vec - m_new) + e
          m_vec = m_new
        return m_vec, s_vec

      m_vec, s_vec = lax.fori_loop(
          0, nblk, p2, (jnp.full((_L,), _NEG), jnp.zeros((_L,), jnp.float32)))
      m_b = jnp.max(m_vec)
      s_b = jnp.sum(s_vec * jnp.exp(m_vec - m_b))
      s_bv = jnp.full((_L,), s_b, jnp.float32)
      inv_s = ones / jnp.where(s_bv > 0.0, s_bv, 1.0)

      def p3(blk, acc):
        @pl.when(multi)
        def _():
          stage_cols(blk)
          pltpu.sync_copy(vatt_hbm.at[cols_v], att_v)
        pltpu.sync_copy(emb_hbm.at[cols_v], rows_v)
        for si in range(_BLK // _L):
          cv, valid = chunk(blk, si)
          cnt = plsc.load_gather(table, [cv])
          cnt = jnp.where(valid, cnt, 1.0)
          a = att_v[pl.ds(si * _L, _L)]
          g = cnt * a
          w = jnp.where(valid, jnp.exp(g - m_b) * (inv_s / cnt), 0.0)
          for j in range(_L):
            wj = w[j]
            tok = si * _L + j
            acc = tuple(
                acc[d] + wj * rows_v[tok, pl.ds(d * _L, _L)]
                for d in range(ND))
        return acc
      acc = lax.fori_loop(
          0, nblk, p3, tuple(jnp.zeros((_L,), jnp.float32) for _ in range(ND)))
      for d in range(ND):
        outb_v[bi, pl.ds(d * _L, _L)] = acc[d]

      def p4(blk, c):
        @pl.when(multi)
        def _():
          stage_cols(blk)
        for si in range(_BLK // _L):
          cv, valid = chunk(blk, si)
          plsc.store_scatter(table, [cv], zeros, mask=valid)
        return c
      lax.fori_loop(0, nblk, p4, 0)
      return c

    lax.fori_loop(0, bags_w, bag_body, 0)
    pltpu.sync_copy(outb_v, out_hbm.at[pl.ds(pl.multiple_of(bag0, 8), bags_w)])

  return k(input_, offs_pad, vocab_att, emb_weight)


def kernel(input_, offsets, emb_weight, proj_w, proj_b, att_h):
  B = offsets.shape[0]
  NNZ = input_.shape[0]
  vocab_att = _att_table_tc(emb_weight, proj_w, proj_b, att_h)
  offs_pad = jnp.concatenate(
      [offsets, jnp.full((_NW * 8,), NNZ, dtype=offsets.dtype)])
  return _pooled_sc(input_, offs_pad, vocab_att, emb_weight, B)
